# baseline (speedup 1.0000x reference)
import jax
import jax.numpy as jnp
from jax.experimental import pallas as pl
from jax.experimental.pallas import tpu as pltpu

INPUT_DIM = 4096
NUM_EXPERTS = 64
TOP_K = 8
TOKENS = 16384
BLOCK_T = 512
NBLOCKS = TOKENS // BLOCK_T
DEPTH = 4


def _copy(x_hbm, xbuf, sem, blk, slot):
    return pltpu.make_async_copy(
        x_hbm.at[pl.ds(blk * BLOCK_T, BLOCK_T), :],
        xbuf.at[slot],
        sem.at[slot],
    )


def _gating_body(x_hbm, w_ref, b_ref, probs_ref, topp_ref, topi_ref,
                 xbuf, sem):
    i = pl.program_id(0)
    slot = jax.lax.rem(i, DEPTH)

    @pl.when(i == 0)
    def _prime():
        for j in range(DEPTH):
            _copy(x_hbm, xbuf, sem, j, j).start()

    nxt = i + DEPTH
    @pl.when((i > 0) & (nxt < NBLOCKS + 1))
    def _prefetch():
        _copy(x_hbm, xbuf, sem, nxt - 1, jax.lax.rem(nxt - 1, DEPTH)).start()

    _copy(x_hbm, xbuf, sem, i, slot).wait()

    x = xbuf[slot]
    w = w_ref[...]
    logits = jax.lax.dot_general(
        w, x,
        dimension_numbers=(((1,), (1,)), ((), ())),
        preferred_element_type=jnp.float32,
    )
    logits = logits + b_ref[...]

    m = jnp.max(logits, axis=0, keepdims=True)
    e = jnp.exp(logits - m)
    s = jnp.sum(e, axis=0, keepdims=True)
    probs_ref[...] = (e / s).T

    row = jax.lax.broadcasted_iota(jnp.int32, logits.shape, 0)
    work = logits
    vals = []
    idxs = []
    for _ in range(TOP_K):
        cur = jnp.max(work, axis=0, keepdims=True)
        hit = work == cur
        idx = jnp.min(jnp.where(hit, row, NUM_EXPERTS), axis=0, keepdims=True)
        vals.append(cur)
        idxs.append(idx)
        work = jnp.where(row == idx, -jnp.inf, work)
    topl = jnp.concatenate(vals, axis=0)
    topv = jnp.exp(topl - m) / s
    total = jnp.sum(topv, axis=0, keepdims=True)
    topp_ref[...] = (topv / total).T
    topi_ref[...] = jnp.concatenate(idxs, axis=0).T


@jax.jit
def kernel(x, W, b):
    b2 = b.reshape(NUM_EXPERTS, 1)
    probs, topp, topi = pl.pallas_call(
        _gating_body,
        grid=(NBLOCKS,),
        in_specs=[
            pl.BlockSpec(memory_space=pl.ANY),
            pl.BlockSpec((NUM_EXPERTS, INPUT_DIM), lambda i: (0, 0)),
            pl.BlockSpec((NUM_EXPERTS, 1), lambda i: (0, 0)),
        ],
        out_specs=[
            pl.BlockSpec((BLOCK_T, NUM_EXPERTS), lambda i: (i, 0)),
            pl.BlockSpec((BLOCK_T, TOP_K), lambda i: (i, 0)),
            pl.BlockSpec((BLOCK_T, TOP_K), lambda i: (i, 0)),
        ],
        out_shape=[
            jax.ShapeDtypeStruct((TOKENS, NUM_EXPERTS), jnp.float32),
            jax.ShapeDtypeStruct((TOKENS, TOP_K), jnp.float32),
            jax.ShapeDtypeStruct((TOKENS, TOP_K), jnp.int32),
        ],
        scratch_shapes=[
            pltpu.VMEM((DEPTH, BLOCK_T, INPUT_DIM), jnp.float32),
            pltpu.SemaphoreType.DMA((DEPTH,)),
        ],
    )(x, W, b2)
    return topp, topi, probs
